# Initial kernel scaffold; baseline (speedup 1.0000x reference)
import jax
import jax.numpy as jnp
from jax.experimental import pallas as pl


def kernel(x, edge_index, edge_attr, batch, tok_emb, bbox_W, bbox_b, rel_emb, Wl, bl, Wr, br, We, att, bo, mlp_W1, mlp_b1, mlp_W2, mlp_b2):
    raise NotImplementedError("write your pallas kernel here")


# speedup vs baseline: 1.3789x; 1.3789x over previous
import functools

import jax
import jax.numpy as jnp
from jax.experimental import pallas as pl

N = 50000
E = 800000
NUM_TOKENS = 151
NUM_RELATIONS = 51
NUM_CLASSES = 8
EMB_DIM = 64
BBOX_DIM = 32
HIDDEN = 96
HEADS = 4
C = HIDDEN // HEADS
IN_DIM = EMB_DIM + BBOX_DIM
NUM_GRAPHS = 64
NEG_SLOPE = 0.2
L = 3

BN = 2000
BE = 8000


def _h0_body(tokg_ref, bbox_ref, bw_ref, bb_ref, out_ref):
    out_ref[:, :EMB_DIM] = tokg_ref[...]
    out_ref[:, EMB_DIM:] = bbox_ref[...] @ bw_ref[...] + bb_ref[...]


def _h0(tok_gathered, bbox, bbox_W, bbox_b):
    grid = (N // BN,)
    return pl.pallas_call(
        _h0_body,
        grid=grid,
        in_specs=[
            pl.BlockSpec((BN, EMB_DIM), lambda i: (i, 0)),
            pl.BlockSpec((BN, 4), lambda i: (i, 0)),
            pl.BlockSpec((4, BBOX_DIM), lambda i: (0, 0)),
            pl.BlockSpec((1, BBOX_DIM), lambda i: (0, 0)),
        ],
        out_specs=pl.BlockSpec((BN, IN_DIM), lambda i: (i, 0)),
        out_shape=jax.ShapeDtypeStruct((N, IN_DIM), jnp.float32),
    )(tok_gathered, bbox, bbox_W, bbox_b.reshape(1, BBOX_DIM))


def _mm2_body(h_ref, wl_ref, bl_ref, wr_ref, br_ref, xl_ref, xr_ref):
    h = h_ref[...]
    xl_ref[...] = h @ wl_ref[...] + bl_ref[...]
    xr_ref[...] = h @ wr_ref[...] + br_ref[...]


def _mm2(h, Wl, bl, Wr, br):
    grid = (N // BN,)
    return pl.pallas_call(
        _mm2_body,
        grid=grid,
        in_specs=[
            pl.BlockSpec((BN, IN_DIM), lambda i: (i, 0)),
            pl.BlockSpec((IN_DIM, HIDDEN), lambda i: (0, 0)),
            pl.BlockSpec((1, HIDDEN), lambda i: (0, 0)),
            pl.BlockSpec((IN_DIM, HIDDEN), lambda i: (0, 0)),
            pl.BlockSpec((1, HIDDEN), lambda i: (0, 0)),
        ],
        out_specs=[
            pl.BlockSpec((BN, HIDDEN), lambda i: (i, 0)),
            pl.BlockSpec((BN, HIDDEN), lambda i: (i, 0)),
        ],
        out_shape=[
            jax.ShapeDtypeStruct((N, HIDDEN), jnp.float32),
            jax.ShapeDtypeStruct((N, HIDDEN), jnp.float32),
        ],
    )(h, Wl, bl.reshape(1, HIDDEN), Wr, br.reshape(1, HIDDEN))


def _edge1_body(xls_ref, xrd_ref, efh_ref, attm_ref, alpha_ref):
    m = xls_ref[...] + xrd_ref[...] + efh_ref[...]
    m = jnp.where(m >= 0, m, NEG_SLOPE * m)
    alpha_ref[...] = m @ attm_ref[...]


def _edge1(xls, xrd, efh, att_mat):
    grid = (E // BE,)
    return pl.pallas_call(
        _edge1_body,
        grid=grid,
        in_specs=[
            pl.BlockSpec((BE, HIDDEN), lambda i: (i, 0)),
            pl.BlockSpec((BE, HIDDEN), lambda i: (i, 0)),
            pl.BlockSpec((BE, HIDDEN), lambda i: (i, 0)),
            pl.BlockSpec((HIDDEN, HEADS), lambda i: (0, 0)),
        ],
        out_specs=pl.BlockSpec((BE, HEADS), lambda i: (i, 0)),
        out_shape=jax.ShapeDtypeStruct((E, HEADS), jnp.float32),
    )(xls, xrd, efh, att_mat)


def _edge2_body(alpha_ref, amaxd_ref, ea_ref):
    ea_ref[...] = jnp.exp(alpha_ref[...] - amaxd_ref[...])


def _edge2(alpha, amaxd):
    grid = (E // BE,)
    return pl.pallas_call(
        _edge2_body,
        grid=grid,
        in_specs=[
            pl.BlockSpec((BE, HEADS), lambda i: (i, 0)),
            pl.BlockSpec((BE, HEADS), lambda i: (i, 0)),
        ],
        out_specs=pl.BlockSpec((BE, HEADS), lambda i: (i, 0)),
        out_shape=jax.ShapeDtypeStruct((E, HEADS), jnp.float32),
    )(alpha, amaxd)


def _edge3_body(xls_ref, ea_ref, asumd_ref, msg_ref):
    w = ea_ref[...] / (asumd_ref[...] + 1e-16)
    xls = xls_ref[...]
    for h in range(HEADS):
        msg_ref[:, h * C:(h + 1) * C] = (
            xls[:, h * C:(h + 1) * C] * w[:, h:h + 1]
        )


def _edge3(xls, ea, asumd):
    grid = (E // BE,)
    return pl.pallas_call(
        _edge3_body,
        grid=grid,
        in_specs=[
            pl.BlockSpec((BE, HIDDEN), lambda i: (i, 0)),
            pl.BlockSpec((BE, HEADS), lambda i: (i, 0)),
            pl.BlockSpec((BE, HEADS), lambda i: (i, 0)),
        ],
        out_specs=pl.BlockSpec((BE, HIDDEN), lambda i: (i, 0)),
        out_shape=jax.ShapeDtypeStruct((E, HIDDEN), jnp.float32),
    )(xls, ea, asumd)


def _biasrelu_body(x_ref, b_ref, out_ref):
    out_ref[...] = jnp.maximum(x_ref[...] + b_ref[...], 0.0)


def _biasrelu(x, b):
    grid = (N // BN,)
    return pl.pallas_call(
        _biasrelu_body,
        grid=grid,
        in_specs=[
            pl.BlockSpec((BN, HIDDEN), lambda i: (i, 0)),
            pl.BlockSpec((1, HIDDEN), lambda i: (0, 0)),
        ],
        out_specs=pl.BlockSpec((BN, HIDDEN), lambda i: (i, 0)),
        out_shape=jax.ShapeDtypeStruct((N, HIDDEN), jnp.float32),
    )(x, b.reshape(1, HIDDEN))


def _reltab_body(rel_ref, we_ref, out_ref):
    out_ref[0] = rel_ref[...] @ we_ref[0]


def _reltab(rel_emb, We):
    grid = (L,)
    return pl.pallas_call(
        _reltab_body,
        grid=grid,
        in_specs=[
            pl.BlockSpec((NUM_RELATIONS, C), lambda i: (0, 0)),
            pl.BlockSpec((1, C, HIDDEN), lambda i: (i, 0, 0)),
        ],
        out_specs=pl.BlockSpec((1, NUM_RELATIONS, HIDDEN), lambda i: (i, 0, 0)),
        out_shape=jax.ShapeDtypeStruct((L, NUM_RELATIONS, HIDDEN), jnp.float32),
    )(rel_emb, We)


def _mlp_body(g_ref, w1_ref, b1_ref, w2_ref, b2_ref, out_ref):
    hdn = jnp.maximum(g_ref[...] @ w1_ref[...] + b1_ref[...], 0.0)
    out_ref[...] = hdn @ w2_ref[...] + b2_ref[...]


def _mlp(g, W1, b1, W2, b2):
    return pl.pallas_call(
        _mlp_body,
        out_shape=jax.ShapeDtypeStruct((NUM_GRAPHS, NUM_CLASSES), jnp.float32),
    )(g, W1, b1.reshape(1, HIDDEN), W2, b2.reshape(1, NUM_CLASSES))


@jax.jit
def _run(x, edge_index, edge_attr, batch, tok_emb, bbox_W, bbox_b, rel_emb,
         Wl, bl, Wr, br, We, att, bo, mlp_W1, mlp_b1, mlp_W2, mlp_b2):
    src = edge_index[0]
    dst = edge_index[1]
    tok_id = x[:, 0].astype(jnp.int32)
    bbox = x[:, 1:5]

    h = _h0(tok_emb[tok_id], bbox, bbox_W, bbox_b)

    rel_tab = _reltab(rel_emb, We)

    att_mats = []
    for i in range(L):
        blocks = [att[i, hh] for hh in range(HEADS)]
        att_mats.append(jax.scipy.linalg.block_diag(
            *[b.reshape(C, 1) for b in blocks]))

    for i in range(L):
        xl, xr = _mm2(h, Wl[i], bl[i], Wr[i], br[i])
        xls = jnp.take(xl, src, axis=0)
        xrd = jnp.take(xr, dst, axis=0)
        efh = jnp.take(rel_tab[i], edge_attr, axis=0)
        alpha = _edge1(xls, xrd, efh, att_mats[i])
        amax = jax.ops.segment_max(alpha, dst, num_segments=N)
        amax = jnp.where(jnp.isfinite(amax), amax, 0.0)
        ea = _edge2(alpha, jnp.take(amax, dst, axis=0))
        asum = jax.ops.segment_sum(ea, dst, num_segments=N)
        msg = _edge3(xls, ea, jnp.take(asum, dst, axis=0))
        out = jax.ops.segment_sum(msg, dst, num_segments=N)
        h = _biasrelu(out, bo[i])

    sums = jax.ops.segment_sum(h, batch, num_segments=NUM_GRAPHS,
                               indices_are_sorted=True)
    cnt = jax.ops.segment_sum(jnp.ones((N, 1), h.dtype), batch,
                              num_segments=NUM_GRAPHS, indices_are_sorted=True)
    g = sums / jnp.maximum(cnt, 1.0)
    return _mlp(g, mlp_W1, mlp_b1, mlp_W2, mlp_b2)


def kernel(x, edge_index, edge_attr, batch, tok_emb, bbox_W, bbox_b, rel_emb,
           Wl, bl, Wr, br, We, att, bo, mlp_W1, mlp_b1, mlp_W2, mlp_b2):
    return _run(x, edge_index, edge_attr, batch, tok_emb, bbox_W, bbox_b,
                rel_emb, Wl, bl, Wr, br, We, att, bo, mlp_W1, mlp_b1,
                mlp_W2, mlp_b2)


# speedup vs baseline: 1.4590x; 1.0581x over previous
import functools

import jax
import jax.numpy as jnp
from jax.experimental import pallas as pl

N = 50000
E = 800000
NUM_TOKENS = 151
NUM_RELATIONS = 51
NUM_CLASSES = 8
EMB_DIM = 64
BBOX_DIM = 32
HIDDEN = 96
HEADS = 4
C = HIDDEN // HEADS
IN_DIM = EMB_DIM + BBOX_DIM
NUM_GRAPHS = 64
NEG_SLOPE = 0.2
L = 3

BN = 2000
BE = 8000


def _h0_body(tokg_ref, bbox_ref, bw_ref, bb_ref, out_ref):
    out_ref[:, :EMB_DIM] = tokg_ref[...]
    out_ref[:, EMB_DIM:] = bbox_ref[...] @ bw_ref[...] + bb_ref[...]


def _h0(tok_gathered, bbox, bbox_W, bbox_b):
    grid = (N // BN,)
    return pl.pallas_call(
        _h0_body,
        grid=grid,
        in_specs=[
            pl.BlockSpec((BN, EMB_DIM), lambda i: (i, 0)),
            pl.BlockSpec((BN, 4), lambda i: (i, 0)),
            pl.BlockSpec((4, BBOX_DIM), lambda i: (0, 0)),
            pl.BlockSpec((1, BBOX_DIM), lambda i: (0, 0)),
        ],
        out_specs=pl.BlockSpec((BN, IN_DIM), lambda i: (i, 0)),
        out_shape=jax.ShapeDtypeStruct((N, IN_DIM), jnp.float32),
    )(tok_gathered, bbox, bbox_W, bbox_b.reshape(1, BBOX_DIM))


def _mm2_body(h_ref, wl_ref, bl_ref, wr_ref, br_ref, xl_ref, xr_ref):
    h = h_ref[...]
    xl_ref[...] = h @ wl_ref[...] + bl_ref[...]
    xr_ref[...] = h @ wr_ref[...] + br_ref[...]


def _mm2(h, Wl, bl, Wr, br):
    grid = (N // BN,)
    return pl.pallas_call(
        _mm2_body,
        grid=grid,
        in_specs=[
            pl.BlockSpec((BN, IN_DIM), lambda i: (i, 0)),
            pl.BlockSpec((IN_DIM, HIDDEN), lambda i: (0, 0)),
            pl.BlockSpec((1, HIDDEN), lambda i: (0, 0)),
            pl.BlockSpec((IN_DIM, HIDDEN), lambda i: (0, 0)),
            pl.BlockSpec((1, HIDDEN), lambda i: (0, 0)),
        ],
        out_specs=[
            pl.BlockSpec((BN, HIDDEN), lambda i: (i, 0)),
            pl.BlockSpec((BN, HIDDEN), lambda i: (i, 0)),
        ],
        out_shape=[
            jax.ShapeDtypeStruct((N, HIDDEN), jnp.float32),
            jax.ShapeDtypeStruct((N, HIDDEN), jnp.float32),
        ],
    )(h, Wl, bl.reshape(1, HIDDEN), Wr, br.reshape(1, HIDDEN))


def _edge1_body(xls_ref, xrd_ref, efh_ref, attm_ref, alpha_ref):
    m = xls_ref[...] + xrd_ref[...] + efh_ref[...]
    m = jnp.where(m >= 0, m, NEG_SLOPE * m)
    alpha_ref[...] = m @ attm_ref[...]


def _edge1(xls, xrd, efh, att_mat):
    grid = (E // BE,)
    return pl.pallas_call(
        _edge1_body,
        grid=grid,
        in_specs=[
            pl.BlockSpec((BE, HIDDEN), lambda i: (i, 0)),
            pl.BlockSpec((BE, HIDDEN), lambda i: (i, 0)),
            pl.BlockSpec((BE, HIDDEN), lambda i: (i, 0)),
            pl.BlockSpec((HIDDEN, HEADS), lambda i: (0, 0)),
        ],
        out_specs=pl.BlockSpec((BE, HEADS), lambda i: (i, 0)),
        out_shape=jax.ShapeDtypeStruct((E, HEADS), jnp.float32),
    )(xls, xrd, efh, att_mat)


def _edge2_body(alpha_ref, amaxd_ref, ea_ref):
    ea_ref[...] = jnp.exp(alpha_ref[...] - amaxd_ref[...])


def _edge2(alpha, amaxd):
    grid = (E // BE,)
    return pl.pallas_call(
        _edge2_body,
        grid=grid,
        in_specs=[
            pl.BlockSpec((BE, HEADS), lambda i: (i, 0)),
            pl.BlockSpec((BE, HEADS), lambda i: (i, 0)),
        ],
        out_specs=pl.BlockSpec((BE, HEADS), lambda i: (i, 0)),
        out_shape=jax.ShapeDtypeStruct((E, HEADS), jnp.float32),
    )(alpha, amaxd)


def _edge3_body(xls_ref, ea_ref, asumd_ref, msg_ref):
    w = ea_ref[...] / (asumd_ref[...] + 1e-16)
    xls = xls_ref[...]
    for h in range(HEADS):
        msg_ref[:, h * C:(h + 1) * C] = (
            xls[:, h * C:(h + 1) * C] * w[:, h:h + 1]
        )


def _edge3(xls, ea, asumd):
    grid = (E // BE,)
    return pl.pallas_call(
        _edge3_body,
        grid=grid,
        in_specs=[
            pl.BlockSpec((BE, HIDDEN), lambda i: (i, 0)),
            pl.BlockSpec((BE, HEADS), lambda i: (i, 0)),
            pl.BlockSpec((BE, HEADS), lambda i: (i, 0)),
        ],
        out_specs=pl.BlockSpec((BE, HIDDEN), lambda i: (i, 0)),
        out_shape=jax.ShapeDtypeStruct((E, HIDDEN), jnp.float32),
    )(xls, ea, asumd)


def _biasrelu_body(x_ref, b_ref, out_ref):
    out_ref[...] = jnp.maximum(x_ref[...] + b_ref[...], 0.0)


def _biasrelu(x, b):
    grid = (N // BN,)
    return pl.pallas_call(
        _biasrelu_body,
        grid=grid,
        in_specs=[
            pl.BlockSpec((BN, HIDDEN), lambda i: (i, 0)),
            pl.BlockSpec((1, HIDDEN), lambda i: (0, 0)),
        ],
        out_specs=pl.BlockSpec((BN, HIDDEN), lambda i: (i, 0)),
        out_shape=jax.ShapeDtypeStruct((N, HIDDEN), jnp.float32),
    )(x, b.reshape(1, HIDDEN))


def _reltab_body(rel_ref, we_ref, out_ref):
    out_ref[0] = rel_ref[...] @ we_ref[0]


def _reltab(rel_emb, We):
    grid = (L,)
    return pl.pallas_call(
        _reltab_body,
        grid=grid,
        in_specs=[
            pl.BlockSpec((NUM_RELATIONS, C), lambda i: (0, 0)),
            pl.BlockSpec((1, C, HIDDEN), lambda i: (i, 0, 0)),
        ],
        out_specs=pl.BlockSpec((1, NUM_RELATIONS, HIDDEN), lambda i: (i, 0, 0)),
        out_shape=jax.ShapeDtypeStruct((L, NUM_RELATIONS, HIDDEN), jnp.float32),
    )(rel_emb, We)


def _mlp_body(g_ref, w1_ref, b1_ref, w2_ref, b2_ref, out_ref):
    hdn = jnp.maximum(g_ref[...] @ w1_ref[...] + b1_ref[...], 0.0)
    out_ref[...] = hdn @ w2_ref[...] + b2_ref[...]


def _mlp(g, W1, b1, W2, b2):
    return pl.pallas_call(
        _mlp_body,
        out_shape=jax.ShapeDtypeStruct((NUM_GRAPHS, NUM_CLASSES), jnp.float32),
    )(g, W1, b1.reshape(1, HIDDEN), W2, b2.reshape(1, NUM_CLASSES))


@jax.jit
def _run(x, edge_index, edge_attr, batch, tok_emb, bbox_W, bbox_b, rel_emb,
         Wl, bl, Wr, br, We, att, bo, mlp_W1, mlp_b1, mlp_W2, mlp_b2):
    order = jnp.argsort(edge_index[1])
    src = edge_index[0][order]
    dst = edge_index[1][order]
    edge_attr = edge_attr[order]
    tok_id = x[:, 0].astype(jnp.int32)
    bbox = x[:, 1:5]

    h = _h0(tok_emb[tok_id], bbox, bbox_W, bbox_b)

    rel_tab = _reltab(rel_emb, We)

    att_mats = []
    for i in range(L):
        blocks = [att[i, hh] for hh in range(HEADS)]
        att_mats.append(jax.scipy.linalg.block_diag(
            *[b.reshape(C, 1) for b in blocks]))

    for i in range(L):
        xl, xr = _mm2(h, Wl[i], bl[i], Wr[i], br[i])
        xls = jnp.take(xl, src, axis=0)
        xrd = jnp.take(xr, dst, axis=0)
        efh = jnp.take(rel_tab[i], edge_attr, axis=0)
        alpha = _edge1(xls, xrd, efh, att_mats[i])
        amax = jax.ops.segment_max(alpha, dst, num_segments=N,
                                   indices_are_sorted=True)
        amax = jnp.where(jnp.isfinite(amax), amax, 0.0)
        ea = _edge2(alpha, jnp.take(amax, dst, axis=0))
        asum = jax.ops.segment_sum(ea, dst, num_segments=N,
                                   indices_are_sorted=True)
        msg = _edge3(xls, ea, jnp.take(asum, dst, axis=0))
        out = jax.ops.segment_sum(msg, dst, num_segments=N,
                                  indices_are_sorted=True)
        h = _biasrelu(out, bo[i])

    sums = jax.ops.segment_sum(h, batch, num_segments=NUM_GRAPHS,
                               indices_are_sorted=True)
    cnt = jax.ops.segment_sum(jnp.ones((N, 1), h.dtype), batch,
                              num_segments=NUM_GRAPHS, indices_are_sorted=True)
    g = sums / jnp.maximum(cnt, 1.0)
    return _mlp(g, mlp_W1, mlp_b1, mlp_W2, mlp_b2)


def kernel(x, edge_index, edge_attr, batch, tok_emb, bbox_W, bbox_b, rel_emb,
           Wl, bl, Wr, br, We, att, bo, mlp_W1, mlp_b1, mlp_W2, mlp_b2):
    return _run(x, edge_index, edge_attr, batch, tok_emb, bbox_W, bbox_b,
                rel_emb, Wl, bl, Wr, br, We, att, bo, mlp_W1, mlp_b1,
                mlp_W2, mlp_b2)


# speedup vs baseline: 1.8615x; 1.2759x over previous
import functools

import jax
import jax.numpy as jnp
from jax.experimental import pallas as pl

N = 50000
E = 800000
NUM_TOKENS = 151
NUM_RELATIONS = 51
NUM_CLASSES = 8
EMB_DIM = 64
BBOX_DIM = 32
HIDDEN = 96
HEADS = 4
C = HIDDEN // HEADS
IN_DIM = EMB_DIM + BBOX_DIM
NUM_GRAPHS = 64
NEG_SLOPE = 0.2
L = 3

BN = 2000
BE = 8000


def _h0_body(tokg_ref, bbox_ref, bw_ref, bb_ref, out_ref):
    out_ref[:, :EMB_DIM] = tokg_ref[...]
    out_ref[:, EMB_DIM:] = bbox_ref[...] @ bw_ref[...] + bb_ref[...]


def _h0(tok_gathered, bbox, bbox_W, bbox_b):
    grid = (N // BN,)
    return pl.pallas_call(
        _h0_body,
        grid=grid,
        in_specs=[
            pl.BlockSpec((BN, EMB_DIM), lambda i: (i, 0)),
            pl.BlockSpec((BN, 4), lambda i: (i, 0)),
            pl.BlockSpec((4, BBOX_DIM), lambda i: (0, 0)),
            pl.BlockSpec((1, BBOX_DIM), lambda i: (0, 0)),
        ],
        out_specs=pl.BlockSpec((BN, IN_DIM), lambda i: (i, 0)),
        out_shape=jax.ShapeDtypeStruct((N, IN_DIM), jnp.float32),
    )(tok_gathered, bbox, bbox_W, bbox_b.reshape(1, BBOX_DIM))


def _mm2_body(h_ref, wl_ref, bl_ref, wr_ref, br_ref, xl_ref, xr_ref):
    h = h_ref[...]
    xl_ref[...] = h @ wl_ref[...] + bl_ref[...]
    xr_ref[...] = h @ wr_ref[...] + br_ref[...]


def _mm2(h, Wl, bl, Wr, br):
    grid = (N // BN,)
    return pl.pallas_call(
        _mm2_body,
        grid=grid,
        in_specs=[
            pl.BlockSpec((BN, IN_DIM), lambda i: (i, 0)),
            pl.BlockSpec((IN_DIM, HIDDEN), lambda i: (0, 0)),
            pl.BlockSpec((1, HIDDEN), lambda i: (0, 0)),
            pl.BlockSpec((IN_DIM, HIDDEN), lambda i: (0, 0)),
            pl.BlockSpec((1, HIDDEN), lambda i: (0, 0)),
        ],
        out_specs=[
            pl.BlockSpec((BN, HIDDEN), lambda i: (i, 0)),
            pl.BlockSpec((BN, HIDDEN), lambda i: (i, 0)),
        ],
        out_shape=[
            jax.ShapeDtypeStruct((N, HIDDEN), jnp.float32),
            jax.ShapeDtypeStruct((N, HIDDEN), jnp.float32),
        ],
    )(h, Wl, bl.reshape(1, HIDDEN), Wr, br.reshape(1, HIDDEN))


def _edge1_body(xls_ref, xrd_ref, efh_ref, attm_ref, alpha_ref):
    m = xls_ref[...] + xrd_ref[...] + efh_ref[...]
    m = jnp.where(m >= 0, m, NEG_SLOPE * m)
    alpha_ref[...] = m @ attm_ref[...]


def _edge1(xls, xrd, efh, att_mat):
    grid = (E // BE,)
    return pl.pallas_call(
        _edge1_body,
        grid=grid,
        in_specs=[
            pl.BlockSpec((BE, HIDDEN), lambda i: (i, 0)),
            pl.BlockSpec((BE, HIDDEN), lambda i: (i, 0)),
            pl.BlockSpec((BE, HIDDEN), lambda i: (i, 0)),
            pl.BlockSpec((HIDDEN, HEADS), lambda i: (0, 0)),
        ],
        out_specs=pl.BlockSpec((BE, HEADS), lambda i: (i, 0)),
        out_shape=jax.ShapeDtypeStruct((E, HEADS), jnp.float32),
    )(xls, xrd, efh, att_mat)


def _edge23_body(xls_ref, alpha_ref, amaxd_ref, out_ref):
    ea = jnp.exp(alpha_ref[...] - amaxd_ref[...])
    xls = xls_ref[...]
    for h in range(HEADS):
        out_ref[:, h * C:(h + 1) * C] = xls[:, h * C:(h + 1) * C] * ea[:, h:h + 1]
    out_ref[:, HIDDEN:] = ea


def _edge23(xls, alpha, amaxd):
    grid = (E // BE,)
    return pl.pallas_call(
        _edge23_body,
        grid=grid,
        in_specs=[
            pl.BlockSpec((BE, HIDDEN), lambda i: (i, 0)),
            pl.BlockSpec((BE, HEADS), lambda i: (i, 0)),
            pl.BlockSpec((BE, HEADS), lambda i: (i, 0)),
        ],
        out_specs=pl.BlockSpec((BE, HIDDEN + HEADS), lambda i: (i, 0)),
        out_shape=jax.ShapeDtypeStruct((E, HIDDEN + HEADS), jnp.float32),
    )(xls, alpha, amaxd)


def _norm_biasrelu_body(u_ref, b_ref, out_ref):
    u = u_ref[...]
    for h in range(HEADS):
        w = 1.0 / (u[:, HIDDEN + h:HIDDEN + h + 1] + 1e-16)
        out_ref[:, h * C:(h + 1) * C] = jnp.maximum(
            u[:, h * C:(h + 1) * C] * w + b_ref[0, h * C:(h + 1) * C], 0.0)


def _norm_biasrelu(u, b):
    grid = (N // BN,)
    return pl.pallas_call(
        _norm_biasrelu_body,
        grid=grid,
        in_specs=[
            pl.BlockSpec((BN, HIDDEN + HEADS), lambda i: (i, 0)),
            pl.BlockSpec((1, HIDDEN), lambda i: (0, 0)),
        ],
        out_specs=pl.BlockSpec((BN, HIDDEN), lambda i: (i, 0)),
        out_shape=jax.ShapeDtypeStruct((N, HIDDEN), jnp.float32),
    )(u, b.reshape(1, HIDDEN))


def _reltab_body(rel_ref, we_ref, out_ref):
    out_ref[0] = rel_ref[...] @ we_ref[0]


def _reltab(rel_emb, We):
    grid = (L,)
    return pl.pallas_call(
        _reltab_body,
        grid=grid,
        in_specs=[
            pl.BlockSpec((NUM_RELATIONS, C), lambda i: (0, 0)),
            pl.BlockSpec((1, C, HIDDEN), lambda i: (i, 0, 0)),
        ],
        out_specs=pl.BlockSpec((1, NUM_RELATIONS, HIDDEN), lambda i: (i, 0, 0)),
        out_shape=jax.ShapeDtypeStruct((L, NUM_RELATIONS, HIDDEN), jnp.float32),
    )(rel_emb, We)


def _mlp_body(g_ref, w1_ref, b1_ref, w2_ref, b2_ref, out_ref):
    hdn = jnp.maximum(g_ref[...] @ w1_ref[...] + b1_ref[...], 0.0)
    out_ref[...] = hdn @ w2_ref[...] + b2_ref[...]


def _mlp(g, W1, b1, W2, b2):
    return pl.pallas_call(
        _mlp_body,
        out_shape=jax.ShapeDtypeStruct((NUM_GRAPHS, NUM_CLASSES), jnp.float32),
    )(g, W1, b1.reshape(1, HIDDEN), W2, b2.reshape(1, NUM_CLASSES))


@jax.jit
def _run(x, edge_index, edge_attr, batch, tok_emb, bbox_W, bbox_b, rel_emb,
         Wl, bl, Wr, br, We, att, bo, mlp_W1, mlp_b1, mlp_W2, mlp_b2):
    order = jnp.argsort(edge_index[1])
    src = edge_index[0][order]
    dst = edge_index[1][order]
    edge_attr = edge_attr[order]
    tok_id = x[:, 0].astype(jnp.int32)
    bbox = x[:, 1:5]

    h = _h0(tok_emb[tok_id], bbox, bbox_W, bbox_b)

    rel_tab = _reltab(rel_emb, We)

    att_mats = []
    for i in range(L):
        blocks = [att[i, hh] for hh in range(HEADS)]
        att_mats.append(jax.scipy.linalg.block_diag(
            *[b.reshape(C, 1) for b in blocks]))

    for i in range(L):
        xl, xr = _mm2(h, Wl[i], bl[i], Wr[i], br[i])
        xls = jnp.take(xl, src, axis=0)
        xrd = jnp.take(xr, dst, axis=0)
        efh = jnp.take(rel_tab[i], edge_attr, axis=0)
        alpha = _edge1(xls, xrd, efh, att_mats[i])
        amax = jax.ops.segment_max(alpha, dst, num_segments=N,
                                   indices_are_sorted=True)
        amax = jnp.where(jnp.isfinite(amax), amax, 0.0)
        comb = _edge23(xls, alpha, jnp.take(amax, dst, axis=0))
        out_u = jax.ops.segment_sum(comb, dst, num_segments=N,
                                    indices_are_sorted=True)
        h = _norm_biasrelu(out_u, bo[i])

    sums = jax.ops.segment_sum(h, batch, num_segments=NUM_GRAPHS,
                               indices_are_sorted=True)
    cnt = jax.ops.segment_sum(jnp.ones((N, 1), h.dtype), batch,
                              num_segments=NUM_GRAPHS, indices_are_sorted=True)
    g = sums / jnp.maximum(cnt, 1.0)
    return _mlp(g, mlp_W1, mlp_b1, mlp_W2, mlp_b2)


def kernel(x, edge_index, edge_attr, batch, tok_emb, bbox_W, bbox_b, rel_emb,
           Wl, bl, Wr, br, We, att, bo, mlp_W1, mlp_b1, mlp_W2, mlp_b2):
    return _run(x, edge_index, edge_attr, batch, tok_emb, bbox_W, bbox_b,
                rel_emb, Wl, bl, Wr, br, We, att, bo, mlp_W1, mlp_b1,
                mlp_W2, mlp_b2)


# speedup vs baseline: 2.1545x; 1.1574x over previous
import functools

import jax
import jax.numpy as jnp
from jax.experimental import pallas as pl

N = 50000
E = 800000
NUM_TOKENS = 151
NUM_RELATIONS = 51
NUM_CLASSES = 8
EMB_DIM = 64
BBOX_DIM = 32
HIDDEN = 96
HEADS = 4
C = HIDDEN // HEADS
IN_DIM = EMB_DIM + BBOX_DIM
NUM_GRAPHS = 64
NEG_SLOPE = 0.2
L = 3

BN = 2000
BE = 8000


def _h0_body(tokg_ref, bbox_ref, bw_ref, bb_ref, out_ref):
    out_ref[:, :EMB_DIM] = tokg_ref[...]
    out_ref[:, EMB_DIM:] = bbox_ref[...] @ bw_ref[...] + bb_ref[...]


def _h0(tok_gathered, bbox, bbox_W, bbox_b):
    grid = (N // BN,)
    return pl.pallas_call(
        _h0_body,
        grid=grid,
        in_specs=[
            pl.BlockSpec((BN, EMB_DIM), lambda i: (i, 0)),
            pl.BlockSpec((BN, 4), lambda i: (i, 0)),
            pl.BlockSpec((4, BBOX_DIM), lambda i: (0, 0)),
            pl.BlockSpec((1, BBOX_DIM), lambda i: (0, 0)),
        ],
        out_specs=pl.BlockSpec((BN, IN_DIM), lambda i: (i, 0)),
        out_shape=jax.ShapeDtypeStruct((N, IN_DIM), jnp.float32),
    )(tok_gathered, bbox, bbox_W, bbox_b.reshape(1, BBOX_DIM))


def _mm2_body(h_ref, wl_ref, bl_ref, wr_ref, br_ref, xl_ref, xr_ref):
    h = h_ref[...]
    xl_ref[...] = h @ wl_ref[...] + bl_ref[...]
    xr_ref[...] = h @ wr_ref[...] + br_ref[...]


def _mm2(h, Wl, bl, Wr, br):
    grid = (N // BN,)
    return pl.pallas_call(
        _mm2_body,
        grid=grid,
        in_specs=[
            pl.BlockSpec((BN, IN_DIM), lambda i: (i, 0)),
            pl.BlockSpec((IN_DIM, HIDDEN), lambda i: (0, 0)),
            pl.BlockSpec((1, HIDDEN), lambda i: (0, 0)),
            pl.BlockSpec((IN_DIM, HIDDEN), lambda i: (0, 0)),
            pl.BlockSpec((1, HIDDEN), lambda i: (0, 0)),
        ],
        out_specs=[
            pl.BlockSpec((BN, HIDDEN), lambda i: (i, 0)),
            pl.BlockSpec((BN, HIDDEN), lambda i: (i, 0)),
        ],
        out_shape=[
            jax.ShapeDtypeStruct((N, HIDDEN), jnp.float32),
            jax.ShapeDtypeStruct((N, HIDDEN), jnp.float32),
        ],
    )(h, Wl, bl.reshape(1, HIDDEN), Wr, br.reshape(1, HIDDEN))


def _edge1_body(xls_ref, xrd_ref, eattr_ref, rel_ref, attm_ref, alpha_ref):
    ids = eattr_ref[...]
    oh = (ids == jax.lax.broadcasted_iota(
        jnp.int32, (ids.shape[0], NUM_RELATIONS), 1)).astype(jnp.float32)
    efh = oh @ rel_ref[...]
    m = xls_ref[...] + xrd_ref[...] + efh
    m = jnp.where(m >= 0, m, NEG_SLOPE * m)
    alpha_ref[...] = m @ attm_ref[...]


def _edge1(xls, xrd, eattr2d, rel_tab_i, att_mat):
    grid = (E // BE,)
    return pl.pallas_call(
        _edge1_body,
        grid=grid,
        in_specs=[
            pl.BlockSpec((BE, HIDDEN), lambda i: (i, 0)),
            pl.BlockSpec((BE, HIDDEN), lambda i: (i, 0)),
            pl.BlockSpec((BE, 1), lambda i: (i, 0)),
            pl.BlockSpec((NUM_RELATIONS, HIDDEN), lambda i: (0, 0)),
            pl.BlockSpec((HIDDEN, HEADS), lambda i: (0, 0)),
        ],
        out_specs=pl.BlockSpec((BE, HEADS), lambda i: (i, 0)),
        out_shape=jax.ShapeDtypeStruct((E, HEADS), jnp.float32),
    )(xls, xrd, eattr2d, rel_tab_i, att_mat)


def _edge23_body(xls_ref, alpha_ref, amaxd_ref, out_ref):
    ea = jnp.exp(alpha_ref[...] - amaxd_ref[...])
    xls = xls_ref[...]
    for h in range(HEADS):
        out_ref[:, h * C:(h + 1) * C] = xls[:, h * C:(h + 1) * C] * ea[:, h:h + 1]
    out_ref[:, HIDDEN:] = ea


def _edge23(xls, alpha, amaxd):
    grid = (E // BE,)
    return pl.pallas_call(
        _edge23_body,
        grid=grid,
        in_specs=[
            pl.BlockSpec((BE, HIDDEN), lambda i: (i, 0)),
            pl.BlockSpec((BE, HEADS), lambda i: (i, 0)),
            pl.BlockSpec((BE, HEADS), lambda i: (i, 0)),
        ],
        out_specs=pl.BlockSpec((BE, HIDDEN + HEADS), lambda i: (i, 0)),
        out_shape=jax.ShapeDtypeStruct((E, HIDDEN + HEADS), jnp.float32),
    )(xls, alpha, amaxd)


def _norm_biasrelu_body(u_ref, b_ref, out_ref):
    u = u_ref[...]
    for h in range(HEADS):
        w = 1.0 / (u[:, HIDDEN + h:HIDDEN + h + 1] + 1e-16)
        out_ref[:, h * C:(h + 1) * C] = jnp.maximum(
            u[:, h * C:(h + 1) * C] * w + b_ref[0, h * C:(h + 1) * C], 0.0)


def _norm_biasrelu(u, b):
    grid = (N // BN,)
    return pl.pallas_call(
        _norm_biasrelu_body,
        grid=grid,
        in_specs=[
            pl.BlockSpec((BN, HIDDEN + HEADS), lambda i: (i, 0)),
            pl.BlockSpec((1, HIDDEN), lambda i: (0, 0)),
        ],
        out_specs=pl.BlockSpec((BN, HIDDEN), lambda i: (i, 0)),
        out_shape=jax.ShapeDtypeStruct((N, HIDDEN), jnp.float32),
    )(u, b.reshape(1, HIDDEN))


def _reltab_body(rel_ref, we_ref, out_ref):
    out_ref[0] = rel_ref[...] @ we_ref[0]


def _reltab(rel_emb, We):
    grid = (L,)
    return pl.pallas_call(
        _reltab_body,
        grid=grid,
        in_specs=[
            pl.BlockSpec((NUM_RELATIONS, C), lambda i: (0, 0)),
            pl.BlockSpec((1, C, HIDDEN), lambda i: (i, 0, 0)),
        ],
        out_specs=pl.BlockSpec((1, NUM_RELATIONS, HIDDEN), lambda i: (i, 0, 0)),
        out_shape=jax.ShapeDtypeStruct((L, NUM_RELATIONS, HIDDEN), jnp.float32),
    )(rel_emb, We)


def _mlp_body(g_ref, w1_ref, b1_ref, w2_ref, b2_ref, out_ref):
    hdn = jnp.maximum(g_ref[...] @ w1_ref[...] + b1_ref[...], 0.0)
    out_ref[...] = hdn @ w2_ref[...] + b2_ref[...]


def _mlp(g, W1, b1, W2, b2):
    return pl.pallas_call(
        _mlp_body,
        out_shape=jax.ShapeDtypeStruct((NUM_GRAPHS, NUM_CLASSES), jnp.float32),
    )(g, W1, b1.reshape(1, HIDDEN), W2, b2.reshape(1, NUM_CLASSES))


@jax.jit
def _run(x, edge_index, edge_attr, batch, tok_emb, bbox_W, bbox_b, rel_emb,
         Wl, bl, Wr, br, We, att, bo, mlp_W1, mlp_b1, mlp_W2, mlp_b2):
    order = jnp.argsort(edge_index[1])
    src = edge_index[0][order]
    dst = edge_index[1][order]
    eattr2d = edge_attr[order].reshape(E, 1)
    tok_id = x[:, 0].astype(jnp.int32)
    bbox = x[:, 1:5]

    h = _h0(tok_emb[tok_id], bbox, bbox_W, bbox_b)

    rel_tab = _reltab(rel_emb, We)

    att_mats = []
    for i in range(L):
        blocks = [att[i, hh] for hh in range(HEADS)]
        att_mats.append(jax.scipy.linalg.block_diag(
            *[b.reshape(C, 1) for b in blocks]))

    for i in range(L):
        xl, xr = _mm2(h, Wl[i], bl[i], Wr[i], br[i])
        xls = jnp.take(xl, src, axis=0)
        xrd = jnp.take(xr, dst, axis=0)
        alpha = _edge1(xls, xrd, eattr2d, rel_tab[i], att_mats[i])
        amax = jax.ops.segment_max(alpha, dst, num_segments=N,
                                   indices_are_sorted=True)
        amax = jnp.where(jnp.isfinite(amax), amax, 0.0)
        comb = _edge23(xls, alpha, jnp.take(amax, dst, axis=0))
        out_u = jax.ops.segment_sum(comb, dst, num_segments=N,
                                    indices_are_sorted=True)
        h = _norm_biasrelu(out_u, bo[i])

    sums = jax.ops.segment_sum(h, batch, num_segments=NUM_GRAPHS,
                               indices_are_sorted=True)
    cnt = jax.ops.segment_sum(jnp.ones((N, 1), h.dtype), batch,
                              num_segments=NUM_GRAPHS, indices_are_sorted=True)
    g = sums / jnp.maximum(cnt, 1.0)
    return _mlp(g, mlp_W1, mlp_b1, mlp_W2, mlp_b2)


def kernel(x, edge_index, edge_attr, batch, tok_emb, bbox_W, bbox_b, rel_emb,
           Wl, bl, Wr, br, We, att, bo, mlp_W1, mlp_b1, mlp_W2, mlp_b2):
    return _run(x, edge_index, edge_attr, batch, tok_emb, bbox_W, bbox_b,
                rel_emb, Wl, bl, Wr, br, We, att, bo, mlp_W1, mlp_b1,
                mlp_W2, mlp_b2)
